# baseline (speedup 1.0000x reference)
import functools

import jax
import jax.numpy as jnp
import numpy as np
from jax import lax
from jax.experimental import pallas as pl
from jax.experimental.pallas import tpu as pltpu
from jax.experimental.pallas import tpu_sc as plsc

B = 16384
N1 = 6
N2 = 146
N2P = 152
EMB = 64
EMBED = 128
OUT = 128
BLK = 8192
NB = B // BLK

NC = 2
NS = 16
NW = NC * NS
BPW = B // NW


def _tab_body(emb1_ref, emb2T_ref, w1_ref, b1_ref, w2_ref, b2_ref, tab_ref):
    t1 = jnp.dot(emb1_ref[...], w1_ref[:EMB, :],
                 preferred_element_type=jnp.float32)
    t2 = lax.dot_general(emb2T_ref[...], w1_ref[EMB:, :],
                         (((0,), (0,)), ((), ())),
                         preferred_element_type=jnp.float32)
    for i in range(N1):
        h = jnp.maximum(t2 + t1[i:i + 1, :] + b1_ref[...], 0.0)
        tab_ref[i * N2P:i * N2P + N2, :] = (
            jnp.dot(h, w2_ref[...], preferred_element_type=jnp.float32)
            + b2_ref[...])


def _argmax_first_f(x, iota_col, n):
    m = jnp.max(x, axis=0, keepdims=True)
    iota = jnp.broadcast_to(iota_col, x.shape)
    return jnp.min(jnp.where(x == m, iota, float(n)), axis=0, keepdims=True)


def _idx_body(ctsT_ref, smlssT_ref, io1_ref, io2_ref, idx_ref):
    i1 = _argmax_first_f(ctsT_ref[...], io1_ref[...], N1)
    i2 = _argmax_first_f(smlssT_ref[...], io2_ref[...], N2)
    idxf = i1 * float(N2P) + i2
    idx_ref[...] = idxf.reshape(BLK // 128, 128).astype(jnp.int32)


@jax.jit
def _tc_stage(cts, smlss, emb1, emb2, W1, b1, W2, b2):
    tab = pl.pallas_call(
        _tab_body,
        in_specs=[
            pl.BlockSpec((N1, EMB), lambda: (0, 0)),
            pl.BlockSpec((EMB, N2), lambda: (0, 0)),
            pl.BlockSpec((EMBED, EMBED), lambda: (0, 0)),
            pl.BlockSpec((1, EMBED), lambda: (0, 0)),
            pl.BlockSpec((EMBED, OUT), lambda: (0, 0)),
            pl.BlockSpec((1, OUT), lambda: (0, 0)),
        ],
        out_specs=pl.BlockSpec((N1 * N2P, OUT), lambda: (0, 0)),
        out_shape=jax.ShapeDtypeStruct((N1 * N2P, OUT), jnp.float32),
    )(emb1, emb2.T, W1, b1.reshape(1, EMBED), W2, b2.reshape(1, OUT))

    idx2 = pl.pallas_call(
        _idx_body,
        grid=(NB,),
        in_specs=[
            pl.BlockSpec((N1, BLK), lambda i: (0, i)),
            pl.BlockSpec((N2, BLK), lambda i: (0, i)),
            pl.BlockSpec((N1, 1), lambda i: (0, 0)),
            pl.BlockSpec((N2, 1), lambda i: (0, 0)),
        ],
        out_specs=pl.BlockSpec((BLK // 128, 128), lambda i: (i, 0)),
        out_shape=jax.ShapeDtypeStruct((B // 128, 128), jnp.int32),
    )(cts.T, smlss.T,
      jnp.asarray(np.arange(N1, dtype=np.float32)).reshape(N1, 1),
      jnp.asarray(np.arange(N2, dtype=np.float32)).reshape(N2, 1))
    return idx2, tab


_sc_mesh = plsc.VectorSubcoreMesh(core_axis_name="c", subcore_axis_name="s")


NCHUNK = 2
CH = BPW // NCHUNK


@jax.jit
@functools.partial(
    pl.kernel, mesh=_sc_mesh,
    out_type=jax.ShapeDtypeStruct((B, OUT), jnp.float32),
    scratch_types=[
        pltpu.VMEM((BPW,), jnp.int32),
    ] + [pltpu.VMEM((CH, OUT), jnp.float32) for _ in range(NCHUNK)]
      + [pltpu.SemaphoreType.DMA for _ in range(2 * NCHUNK)],
)
def _sc_gather(tab_hbm, idx_hbm, out_hbm, idx_v, *bufs_and_sems):
    rows = bufs_and_sems[:NCHUNK]
    gsem = bufs_and_sems[NCHUNK:2 * NCHUNK]
    psem = bufs_and_sems[2 * NCHUNK:]
    wid = lax.axis_index("s") * NC + lax.axis_index("c")
    base = wid * BPW
    pltpu.sync_copy(idx_hbm.at[pl.ds(base, BPW)], idx_v)
    gats = [pltpu.async_copy(tab_hbm.at[idx_v.at[pl.ds(k * CH, CH)]],
                             rows[k], gsem[k]) for k in range(NCHUNK)]
    puts = []
    for k in range(NCHUNK):
        gats[k].wait()
        puts.append(pltpu.async_copy(
            rows[k], out_hbm.at[pl.ds(base + k * CH, CH)], psem[k]))
    for p in puts:
        p.wait()


def kernel(cts, smlss, emb1, emb2, W1, b1, W2, b2):
    idx2, tab = _tc_stage(cts, smlss, emb1, emb2, W1, b1, W2, b2)
    return _sc_gather(tab, idx2.reshape(B))


# speedup vs baseline: 1.0806x; 1.0806x over previous
import functools

import jax
import jax.numpy as jnp
import numpy as np
from jax import lax
from jax.experimental import pallas as pl
from jax.experimental.pallas import tpu as pltpu
from jax.experimental.pallas import tpu_sc as plsc

B = 16384
N1 = 6
N2 = 146
N2P = 152
EMB = 64
EMBED = 128
OUT = 128
BLK = 4096
NB = B // BLK

NC = 2
NS = 16
NW = NC * NS
BPW = B // NW


def _tab_body(emb1_ref, emb2T_ref, w1_ref, b1_ref, w2_ref, b2_ref, tab_ref):
    t1 = jnp.dot(emb1_ref[...], w1_ref[:EMB, :],
                 preferred_element_type=jnp.float32)
    t2 = lax.dot_general(emb2T_ref[...], w1_ref[EMB:, :],
                         (((0,), (0,)), ((), ())),
                         preferred_element_type=jnp.float32)
    for i in range(N1):
        h = jnp.maximum(t2 + t1[i:i + 1, :] + b1_ref[...], 0.0)
        tab_ref[i * N2P:i * N2P + N2, :] = (
            jnp.dot(h, w2_ref[...], preferred_element_type=jnp.float32)
            + b2_ref[...])


def _argmax_first_f(x, iota_col, n):
    m = jnp.max(x, axis=0, keepdims=True)
    iota = jnp.broadcast_to(iota_col, x.shape)
    return jnp.min(jnp.where(x == m, iota, float(n)), axis=0, keepdims=True)


def _idx_body(ctsT_ref, smlssT_ref, io1_ref, io2_ref, emb1_ref, emb2T_ref,
              w1_ref, b1_ref, w2_ref, b2_ref, idx_ref, tab_ref):
    i1 = _argmax_first_f(ctsT_ref[...], io1_ref[...], N1)
    i2 = _argmax_first_f(smlssT_ref[...], io2_ref[...], N2)
    idxf = i1 * float(N2P) + i2
    idx_ref[...] = idxf.reshape(BLK // 128, 128).astype(jnp.int32)

    @pl.when(pl.program_id(0) == NB - 1)
    def _build_table():
        _tab_body(emb1_ref, emb2T_ref, w1_ref, b1_ref, w2_ref, b2_ref,
                  tab_ref)


@jax.jit
def _tc_stage(cts, smlss, emb1, emb2, W1, b1, W2, b2):
    idx2, tab = pl.pallas_call(
        _idx_body,
        grid=(NB,),
        in_specs=[
            pl.BlockSpec((N1, BLK), lambda i: (0, i)),
            pl.BlockSpec((N2, BLK), lambda i: (0, i)),
            pl.BlockSpec((N1, 1), lambda i: (0, 0)),
            pl.BlockSpec((N2, 1), lambda i: (0, 0)),
            pl.BlockSpec((N1, EMB), lambda i: (0, 0)),
            pl.BlockSpec((EMB, N2), lambda i: (0, 0)),
            pl.BlockSpec((EMBED, EMBED), lambda i: (0, 0)),
            pl.BlockSpec((1, EMBED), lambda i: (0, 0)),
            pl.BlockSpec((EMBED, OUT), lambda i: (0, 0)),
            pl.BlockSpec((1, OUT), lambda i: (0, 0)),
        ],
        out_specs=[
            pl.BlockSpec((BLK // 128, 128), lambda i: (i, 0)),
            pl.BlockSpec((N1 * N2P, OUT), lambda i: (0, 0)),
        ],
        out_shape=[
            jax.ShapeDtypeStruct((B // 128, 128), jnp.int32),
            jax.ShapeDtypeStruct((N1 * N2P, OUT), jnp.float32),
        ],
    )(cts.T, smlss.T,
      jnp.asarray(np.arange(N1, dtype=np.float32)).reshape(N1, 1),
      jnp.asarray(np.arange(N2, dtype=np.float32)).reshape(N2, 1),
      emb1, emb2.T, W1, b1.reshape(1, EMBED), W2, b2.reshape(1, OUT))
    return idx2, tab


_sc_mesh = plsc.VectorSubcoreMesh(core_axis_name="c", subcore_axis_name="s")


NCHUNK = 2
CH = BPW // NCHUNK


@jax.jit
@functools.partial(
    pl.kernel, mesh=_sc_mesh,
    out_type=jax.ShapeDtypeStruct((B, OUT), jnp.float32),
    scratch_types=[
        pltpu.VMEM((BPW,), jnp.int32),
    ] + [pltpu.VMEM((CH, OUT), jnp.float32) for _ in range(NCHUNK)]
      + [pltpu.SemaphoreType.DMA for _ in range(2 * NCHUNK)],
)
def _sc_gather(tab_hbm, idx_hbm, out_hbm, idx_v, *bufs_and_sems):
    rows = bufs_and_sems[:NCHUNK]
    gsem = bufs_and_sems[NCHUNK:2 * NCHUNK]
    psem = bufs_and_sems[2 * NCHUNK:]
    wid = lax.axis_index("s") * NC + lax.axis_index("c")
    base = wid * BPW
    pltpu.sync_copy(idx_hbm.at[pl.ds(base, BPW)], idx_v)
    gats = [pltpu.async_copy(tab_hbm.at[idx_v.at[pl.ds(k * CH, CH)]],
                             rows[k], gsem[k]) for k in range(NCHUNK)]
    puts = []
    for k in range(NCHUNK):
        gats[k].wait()
        puts.append(pltpu.async_copy(
            rows[k], out_hbm.at[pl.ds(base + k * CH, CH)], psem[k]))
    for p in puts:
        p.wait()


def kernel(cts, smlss, emb1, emb2, W1, b1, W2, b2):
    idx2, tab = _tc_stage(cts, smlss, emb1, emb2, W1, b1, W2, b2)
    return _sc_gather(tab, idx2.reshape(B))
